# baseline (speedup 1.0000x reference)
import functools

import jax
import jax.numpy as jnp
from jax import lax
from jax.experimental import pallas as pl
from jax.experimental.pallas import tpu as pltpu
from jax.experimental.pallas import tpu_sc as plsc

_B = 16384
_DIM = 32
_NW = 32
_BPW = _B // _NW
_NBUF = 10
_G = 16

_mesh = plsc.VectorSubcoreMesh(core_axis_name="c", subcore_axis_name="s")


@functools.partial(
    pl.kernel,
    mesh=_mesh,
    out_type=jax.ShapeDtypeStruct((_B, 4 * _DIM), jnp.float32),
    compiler_params=pltpu.CompilerParams(needs_layout_passes=False),
    scratch_types=[
        pltpu.VMEM((_BPW + _G,), jnp.int32),
        pltpu.VMEM((_BPW,), jnp.int32),
        pltpu.VMEM((_BPW,), jnp.int32),
        pltpu.VMEM((_BPW,), jnp.int32),
        pltpu.VMEM((_DIM, 128), jnp.float32),
        pltpu.VMEM((_DIM, 128), jnp.float32),
        pltpu.VMEM((_DIM, 128), jnp.float32),
        [pltpu.VMEM((_DIM, 128), jnp.float32)] * _NBUF,
        pltpu.VMEM((_BPW, 4 * _DIM), jnp.float32),
        [pltpu.SemaphoreType.DMA] * _NBUF,
    ],
)
def _emb_concat(cid, age, col, pg, utT, aT, cT, pT, out,
                i0, i1, i2, i3, av, cv, pv, bufs, ov, sems):
    wid = lax.axis_index("s") * 2 + lax.axis_index("c")
    base = wid * _BPW

    pltpu.sync_copy(cid.at[pl.ds(base, _BPW)], i0.at[pl.ds(0, _BPW)])
    pltpu.sync_copy(age.at[pl.ds(base, _BPW)], i1)
    pltpu.sync_copy(col.at[pl.ds(base, _BPW)], i2)
    pltpu.sync_copy(pg.at[pl.ds(base, _BPW)], i3)
    pltpu.sync_copy(aT, av)
    pltpu.sync_copy(cT, cv)
    pltpu.sync_copy(pT, pv)

    rows_lo = jax.lax.iota(jnp.int32, 16)
    rows_hi = rows_lo + 16

    def fetch(x, b):
        c0 = pl.multiple_of((x // 128) * 128, 128)
        pltpu.async_copy(utT.at[:, pl.ds(c0, 128)], bufs[b], sems[b])

    def gather_col(tbl, lane):
        l16 = jnp.full((16,), lane, jnp.int32)
        lo = plsc.load_gather(tbl, [rows_lo, l16])
        hi = plsc.load_gather(tbl, [rows_hi, l16])
        return lo, hi

    xv0 = i0[pl.ds(0, _G)]
    for b in range(_NBUF):
        fetch(xv0[b], b)

    def body(g, carry):
        xv = i0[pl.ds(g * _G, _G)]
        xnv = i0[pl.ds(g * _G + _G, _G)]
        av_ = i1[pl.ds(g * _G, _G)]
        cv_ = i2[pl.ds(g * _G, _G)]
        pv_ = i3[pl.ds(g * _G, _G)]
        for b in range(_G):
            j = g * _G + b
            slot = b % _NBUF
            pltpu.make_async_copy(
                utT.at[:, pl.ds(0, 128)], bufs[slot], sems[slot]).wait()
            lo, hi = gather_col(bufs[slot], xv[b] % 128)
            ov[j, pl.ds(0, 16)] = lo
            ov[j, pl.ds(16, 16)] = hi
            lo, hi = gather_col(av, av_[b])
            ov[j, pl.ds(32, 16)] = lo
            ov[j, pl.ds(48, 16)] = hi
            lo, hi = gather_col(cv, cv_[b])
            ov[j, pl.ds(64, 16)] = lo
            ov[j, pl.ds(80, 16)] = hi
            lo, hi = gather_col(pv, pv_[b])
            ov[j, pl.ds(96, 16)] = lo
            ov[j, pl.ds(112, 16)] = hi
            xn = xv[b + _NBUF] if b < _G - _NBUF else xnv[b - (_G - _NBUF)]

            @pl.when(j + _NBUF < _BPW)
            def _():
                fetch(xn, slot)

        return carry

    lax.fori_loop(0, _BPW // _G, body, 0)
    pltpu.sync_copy(ov, out.at[pl.ds(base, _BPW)])


def kernel(customer_id, age, colour_group_name, product_group_name,
           user_table, age_table, colour_table, prod_group_table):
    cid = jnp.asarray(customer_id, jnp.int32)
    a = jnp.asarray(age, jnp.int32)
    c = jnp.asarray(colour_group_name, jnp.int32)
    p = jnp.asarray(product_group_name, jnp.int32)
    utT = user_table.T
    aT = jnp.pad(age_table.T, ((0, 0), (0, 128 - age_table.shape[0])))
    cT = jnp.pad(colour_table.T, ((0, 0), (0, 128 - colour_table.shape[0])))
    pT = jnp.pad(prod_group_table.T,
                 ((0, 0), (0, 128 - prod_group_table.shape[0])))
    return _emb_concat(cid, a, c, p, utT, aT, cT, pT)


# speedup vs baseline: 1.1022x; 1.1022x over previous
import functools

import jax
import jax.numpy as jnp
from jax import lax
from jax.experimental import pallas as pl
from jax.experimental.pallas import tpu as pltpu
from jax.experimental.pallas import tpu_sc as plsc

_B = 16384
_DIM = 32
_NW = 32
_BPW = _B // _NW
_NBUF = 8
_G = 16

_mesh = plsc.VectorSubcoreMesh(core_axis_name="c", subcore_axis_name="s")


@functools.partial(
    pl.kernel,
    mesh=_mesh,
    out_type=jax.ShapeDtypeStruct((_B, 4 * _DIM), jnp.float32),
    compiler_params=pltpu.CompilerParams(needs_layout_passes=False),
    scratch_types=[
        pltpu.VMEM((_BPW + _G,), jnp.int32),
        pltpu.VMEM((_BPW,), jnp.int32),
        pltpu.VMEM((_BPW,), jnp.int32),
        pltpu.VMEM((_BPW,), jnp.int32),
        pltpu.VMEM((_DIM, 128), jnp.float32),
        pltpu.VMEM((_DIM, 128), jnp.float32),
        pltpu.VMEM((_DIM, 128), jnp.float32),
        [pltpu.VMEM((_DIM, 128), jnp.float32)] * _NBUF,
        pltpu.VMEM((_BPW, 4 * _DIM), jnp.float32),
        [pltpu.SemaphoreType.DMA] * _NBUF,
    ],
)
def _emb_concat(cid, age, col, pg, utT, aT, cT, pT, out,
                i0, i1, i2, i3, av, cv, pv, bufs, ov, sems):
    wid = lax.axis_index("s") * 2 + lax.axis_index("c")
    base = wid * _BPW

    pltpu.sync_copy(cid.at[pl.ds(base, _BPW)], i0.at[pl.ds(0, _BPW)])
    pltpu.sync_copy(age.at[pl.ds(base, _BPW)], i1)
    pltpu.sync_copy(col.at[pl.ds(base, _BPW)], i2)
    pltpu.sync_copy(pg.at[pl.ds(base, _BPW)], i3)
    pltpu.sync_copy(aT, av)
    pltpu.sync_copy(cT, cv)
    pltpu.sync_copy(pT, pv)

    rows_lo = jax.lax.iota(jnp.int32, 16)
    rows_hi = rows_lo + 16

    def fetch(x, b):
        c0 = pl.multiple_of((x // 128) * 128, 128)
        for f in range(4):
            pltpu.async_copy(utT.at[pl.ds(8 * f, 8), pl.ds(c0, 128)],
                             bufs[b].at[pl.ds(8 * f, 8)], sems[b])

    def gather_col(tbl, lane):
        l16 = jnp.full((16,), lane, jnp.int32)
        lo = plsc.load_gather(tbl, [rows_lo, l16])
        hi = plsc.load_gather(tbl, [rows_hi, l16])
        return lo, hi

    xv0 = i0[pl.ds(0, _G)]
    for b in range(_NBUF):
        fetch(xv0[b], b)

    def body(g, carry):
        xv = i0[pl.ds(g * _G, _G)]
        xnv = i0[pl.ds(g * _G + _G, _G)]
        av_ = i1[pl.ds(g * _G, _G)]
        cv_ = i2[pl.ds(g * _G, _G)]
        pv_ = i3[pl.ds(g * _G, _G)]
        for b in range(_G):
            j = g * _G + b
            slot = b % _NBUF
            for f in range(4):
                pltpu.make_async_copy(
                    utT.at[pl.ds(8 * f, 8), pl.ds(0, 128)],
                    bufs[slot].at[pl.ds(8 * f, 8)], sems[slot]).wait()
            lo, hi = gather_col(bufs[slot], xv[b] % 128)
            ov[j, pl.ds(0, 16)] = lo
            ov[j, pl.ds(16, 16)] = hi
            lo, hi = gather_col(av, av_[b])
            ov[j, pl.ds(32, 16)] = lo
            ov[j, pl.ds(48, 16)] = hi
            lo, hi = gather_col(cv, cv_[b])
            ov[j, pl.ds(64, 16)] = lo
            ov[j, pl.ds(80, 16)] = hi
            lo, hi = gather_col(pv, pv_[b])
            ov[j, pl.ds(96, 16)] = lo
            ov[j, pl.ds(112, 16)] = hi
            xn = xv[b + _NBUF] if b < _G - _NBUF else xnv[b - (_G - _NBUF)]

            @pl.when(j + _NBUF < _BPW)
            def _():
                fetch(xn, slot)

        return carry

    lax.fori_loop(0, _BPW // _G, body, 0)
    pltpu.sync_copy(ov, out.at[pl.ds(base, _BPW)])


def kernel(customer_id, age, colour_group_name, product_group_name,
           user_table, age_table, colour_table, prod_group_table):
    cid = jnp.asarray(customer_id, jnp.int32)
    a = jnp.asarray(age, jnp.int32)
    c = jnp.asarray(colour_group_name, jnp.int32)
    p = jnp.asarray(product_group_name, jnp.int32)
    utT = user_table.T
    aT = jnp.pad(age_table.T, ((0, 0), (0, 128 - age_table.shape[0])))
    cT = jnp.pad(colour_table.T, ((0, 0), (0, 128 - colour_table.shape[0])))
    pT = jnp.pad(prod_group_table.T,
                 ((0, 0), (0, 128 - prod_group_table.shape[0])))
    return _emb_concat(cid, a, c, p, utT, aT, cT, pT)
